# baseline (speedup 1.0000x reference)
import functools

import jax
import jax.numpy as jnp
from jax import lax
from jax.experimental import pallas as pl
from jax.experimental.pallas import tpu as pltpu
from jax.experimental.pallas import tpu_sc as plsc

NUM_USER = 2000
NUM_ITEM = 8000
DIM = 128
N = NUM_USER + NUM_ITEM
E = 320000
NC, NS = 2, 16
NW = NC * NS
EPT = E // NW
CH = 80
NCH = EPT // CH
NB = 3
K = 1
NR = 4
NI = 8
HD = DIM // NC
EPTS = E // NS
NCHS = EPTS // CH
N_PAD = 10112
RPT = N_PAD // NS

_MESH = plsc.VectorSubcoreMesh(
    core_axis_name="c", subcore_axis_name="s", num_cores=NC, num_subcores=NS
)


@functools.partial(
    pl.kernel,
    out_type=jax.ShapeDtypeStruct((NC * N,), jnp.float32),
    mesh=_MESH,
    scratch_types=[
        pltpu.VMEM((CH,), jnp.float32),
        pltpu.VMEM((N,), jnp.float32),
        pltpu.VMEM_SHARED((N,), jnp.float32),
    ]
    + [pltpu.VMEM((CH,), jnp.int32) for _ in range(NB)]
    + [pltpu.SemaphoreType.DMA for _ in range(2 * NB)],
)
def _deg_kernel(dst_hbm, out_hbm, ones_v, stage_v, acc, *ring):
    idxv = ring[:NB]
    semd = ring[NB:2 * NB]
    sems = ring[2 * NB:]
    c = lax.axis_index("c")
    s = lax.axis_index("s")
    w = c * NS + s
    one16 = jnp.full((16,), 1.0, jnp.float32)
    for i in range(CH // 16):
        ones_v[pl.ds(i * 16, 16)] = one16

    @pl.when(s == 0)
    def _():
        def fill(i, carry):
            stage_v[pl.ds(i * 16, 16)] = one16
            return carry
        lax.fori_loop(0, N // 16, fill, 0)
        pltpu.sync_copy(stage_v, acc)

    plsc.subcore_barrier()

    def d_start(b, j):
        off = pl.multiple_of(w * EPT + j * CH, 8)
        pltpu.async_copy(dst_hbm.at[pl.ds(off, CH)], idxv[b], semd[b])

    def d_wait(b):
        pltpu.make_async_copy(dst_hbm.at[pl.ds(0, CH)], idxv[b], semd[b]).wait()

    def s_start(b):
        pltpu.async_copy(ones_v, acc.at[idxv[b]], sems[b], add=True)

    def s_wait(b):
        pltpu.make_async_copy(ones_v, acc.at[idxv[b]], sems[b]).wait()

    def chunk_body(j, b, prefetch_wait):
        jn = jnp.minimum(j + K, NCH - 1)
        bn = (b + K) % NB
        if prefetch_wait:
            s_wait(bn)
        d_start(bn, jn)
        d_wait(b)
        s_start(b)

    d_start(0, 0)
    for j in range(NB):
        chunk_body(j, j % NB, prefetch_wait=(j + K >= NB))

    def outer(i, carry):
        for b in range(NB):
            chunk_body(i * NB + b, b, prefetch_wait=True)
        return carry

    lax.fori_loop(1, (NCH - 2) // NB, outer, 0)
    chunk_body(NCH - 2, (NCH - 2) % NB, prefetch_wait=True)
    chunk_body(NCH - 1, (NCH - 1) % NB, prefetch_wait=True)
    s_wait((NCH - 2) % NB)
    s_wait((NCH - 1) % NB)
    d_wait(NCH % NB)
    plsc.subcore_barrier()

    @pl.when(s == 0)
    def _():
        pltpu.sync_copy(acc, stage_v)
        pltpu.sync_copy(stage_v, out_hbm.at[pl.ds(pl.multiple_of(c * N, 8), N)])


@functools.partial(
    pl.kernel,
    out_type=jax.ShapeDtypeStruct((NC, N_PAD, HD), jnp.float32),
    mesh=_MESH,
    scratch_types=[
        pltpu.VMEM_SHARED((N_PAD, HD), jnp.float32),
        pltpu.VMEM_SHARED((N_PAD, HD), jnp.float32),
    ]
    + [pltpu.VMEM((CH, HD), jnp.float32) for _ in range(NR)]
    + [pltpu.VMEM((CH,), jnp.int32) for _ in range(NI)]
    + [pltpu.VMEM((CH,), jnp.int32) for _ in range(NI)]
    + [pltpu.SemaphoreType.DMA for _ in range(2 * NR + 2 * NI)],
    compiler_params=pltpu.CompilerParams(use_tc_tiling_on_sc=False),
)
def _scatter_kernel(src_hbm, dst_hbm, y_hbm, out_hbm, yspm, acc, *ring):
    rows = ring[:NR]
    o = NR
    srcv = ring[o:o + NI]; o += NI
    dstv = ring[o:o + NI]; o += NI
    semg = ring[o:o + NR]; o += NR
    sems = ring[o:o + NR]; o += NR
    semi = ring[o:o + NI]; o += NI
    semd = ring[o:o + NI]
    c = lax.axis_index("c")
    s = lax.axis_index("s")
    r0 = pl.multiple_of(s * RPT, 8)
    pltpu.sync_copy(y_hbm.at[c, pl.ds(r0, RPT)], yspm.at[pl.ds(r0, RPT)])
    pltpu.sync_copy(y_hbm.at[c, pl.ds(r0, RPT)], acc.at[pl.ds(r0, RPT)])
    plsc.subcore_barrier()

    def eoff(j):
        return pl.multiple_of(s * EPTS + j * CH, 8)

    def i_start(j, m):
        b = m % NI
        pltpu.async_copy(src_hbm.at[pl.ds(eoff(j), CH)], srcv[b], semi[b])

    def i_wait(m):
        b = m % NI
        pltpu.make_async_copy(src_hbm.at[pl.ds(0, CH)], srcv[b], semi[b]).wait()

    def d_start(j, m):
        b = m % NI
        pltpu.async_copy(dst_hbm.at[pl.ds(eoff(j), CH)], dstv[b], semd[b])

    def d_wait(m):
        b = m % NI
        pltpu.make_async_copy(dst_hbm.at[pl.ds(0, CH)], dstv[b], semd[b]).wait()

    def g_start(m):
        pltpu.async_copy(yspm.at[srcv[m % NI]], rows[m % NR], semg[m % NR])

    def g_wait(m):
        b = m % NR
        pltpu.make_async_copy(yspm.at[srcv[0]], rows[b], semg[b]).wait()

    def s_start(m):
        pltpu.async_copy(rows[m % NR], acc.at[dstv[m % NI]], sems[m % NR], add=True)

    def s_wait(m):
        b = m % NR
        pltpu.make_async_copy(rows[b], acc.at[dstv[0]], sems[b]).wait()

    def chunk_body(j, m, prev3=True, next2=True, next1=True):
        if prev3:
            s_wait(m - 3)
        if next2:
            i_start(j + 2, m + 2)
            d_start(j + 2, m + 2)
        if next1:
            i_wait(m + 1)
            g_start(m + 1)
        g_wait(m)
        d_wait(m)
        s_start(m)

    i_start(0, 0); d_start(0, 0); i_start(1, 1); d_start(1, 1)
    i_wait(0); g_start(0)
    for j in range(5):
        chunk_body(j, j, prev3=(j >= 3))

    def outer(i, carry):
        j0 = 5 + i * 8
        for t in range(8):
            chunk_body(j0 + t, 5 + t)
        return carry

    lax.fori_loop(0, (NCHS - 10) // 8, outer, 0)
    for j in range(NCHS - 5, NCHS):
        chunk_body(j, j, next2=(j + 2 < NCHS), next1=(j + 1 < NCHS))
    for m in (NCHS - 3, NCHS - 2, NCHS - 1):
        s_wait(m)
    plsc.subcore_barrier()
    pltpu.sync_copy(acc.at[pl.ds(r0, RPT)], out_hbm.at[c, pl.ds(r0, RPT)])


def _mlp_body(f_ref, p_ref, w1_ref, b1_ref, w2_ref, b2_ref, out_ref):
    h0 = jnp.dot(f_ref[...], w1_ref[...], preferred_element_type=jnp.float32)
    h0 = h0 + b1_ref[...]
    h0 = jnp.where(h0 >= 0, h0, 0.01 * h0)
    out_ref[0:NUM_USER, :] = p_ref[...]
    out_ref[NUM_USER:N, :] = (
        jnp.dot(h0, w2_ref[...], preferred_element_type=jnp.float32) + b2_ref[...]
    )


def _dinv_col(deg_ref):
    dsum = deg_ref[:, 0:1] + deg_ref[:, 1:2] - 1.0
    return lax.rsqrt(dsum)


def _write_halves(y_ref, y):
    zpad = jnp.zeros((N_PAD - N, HD), jnp.float32)
    y_ref[0, 0:N, :] = y[:, 0:HD]
    y_ref[0, N:N_PAD, :] = zpad
    y_ref[1, 0:N, :] = y[:, HD:DIM]
    y_ref[1, N:N_PAD, :] = zpad


def _cat_halves(s_ref):
    return jnp.concatenate([s_ref[0, 0:N, :], s_ref[1, 0:N, :]], axis=1)


def _prep_body(x_ref, deg_ref, w_ref, xn_ref, y_ref):
    x = x_ref[...]
    n2 = jnp.sum(x * x, axis=1, keepdims=True)
    nrm = jnp.maximum(jnp.sqrt(n2), 1e-12)
    xn = x / nrm
    xn_ref[...] = xn
    y = jnp.dot(xn, w_ref[...], preferred_element_type=jnp.float32) * _dinv_col(deg_ref)
    _write_halves(y_ref, y)


def _mid_body(s_ref, deg_ref, w_ref, b_ref, h_ref, y2_ref):
    dinv = _dinv_col(deg_ref)
    h = dinv * _cat_halves(s_ref) + b_ref[...]
    h_ref[...] = h
    y2 = jnp.dot(h, w_ref[...], preferred_element_type=jnp.float32) * dinv
    _write_halves(y2_ref, y2)


def _fin_body(s_ref, h_ref, xn_ref, deg_ref, b_ref, out_ref):
    dinv = _dinv_col(deg_ref)
    h1 = dinv * _cat_halves(s_ref) + b_ref[...]
    out_ref[...] = xn_ref[...] + h_ref[...] + h1


def _f32(*shape):
    return jax.ShapeDtypeStruct(shape, jnp.float32)


def kernel(edge_index, features, preference, W_mlp, b_mlp, W_mlp1, b_mlp1, W_conv, b_conv):
    src_f = edge_index[0].astype(jnp.int32)
    dst_f = edge_index[1].astype(jnp.int32)

    deg2 = _deg_kernel(dst_f).reshape(NC, N)
    degT = deg2.T

    xcat = pl.pallas_call(_mlp_body, out_shape=_f32(N, DIM))(
        features, preference, W_mlp.T, b_mlp.reshape(1, -1), W_mlp1.T,
        b_mlp1.reshape(1, -1)
    )

    xn, y1 = pl.pallas_call(
        _prep_body, out_shape=(_f32(N, DIM), _f32(NC, N_PAD, HD))
    )(xcat, degT, W_conv)

    s1 = _scatter_kernel(src_f, dst_f, y1)
    h, y2 = pl.pallas_call(
        _mid_body, out_shape=(_f32(N, DIM), _f32(NC, N_PAD, HD))
    )(s1, degT, W_conv, b_conv.reshape(1, -1))

    s2 = _scatter_kernel(src_f, dst_f, y2)
    x_hat = pl.pallas_call(_fin_body, out_shape=_f32(N, DIM))(
        s2, h, xn, degT, b_conv.reshape(1, -1)
    )
    return (x_hat, preference)


# speedup vs baseline: 1.1973x; 1.1973x over previous
import functools

import jax
import jax.numpy as jnp
from jax import lax
from jax.experimental import pallas as pl
from jax.experimental.pallas import tpu as pltpu
from jax.experimental.pallas import tpu_sc as plsc

NUM_USER = 2000
NUM_ITEM = 8000
DIM = 128
N = NUM_USER + NUM_ITEM
E = 320000
NC, NS = 2, 16
NW = NC * NS
EPT = E // NW
CH = 80
NCH = EPT // CH
NB = 3
K = 1
NR = 4
NI = 8
N_PAD = 10112
RPT = N_PAD // NS

_MESH = plsc.VectorSubcoreMesh(
    core_axis_name="c", subcore_axis_name="s", num_cores=NC, num_subcores=NS
)


@functools.partial(
    pl.kernel,
    out_type=jax.ShapeDtypeStruct((NC * N,), jnp.float32),
    mesh=_MESH,
    scratch_types=[
        pltpu.VMEM((CH,), jnp.float32),
        pltpu.VMEM((N,), jnp.float32),
        pltpu.VMEM_SHARED((N,), jnp.float32),
    ]
    + [pltpu.VMEM((CH,), jnp.int32) for _ in range(NB)]
    + [pltpu.SemaphoreType.DMA for _ in range(2 * NB)],
)
def _deg_kernel(dst_hbm, out_hbm, ones_v, stage_v, acc, *ring):
    idxv = ring[:NB]
    semd = ring[NB:2 * NB]
    sems = ring[2 * NB:]
    c = lax.axis_index("c")
    s = lax.axis_index("s")
    w = c * NS + s
    one16 = jnp.full((16,), 1.0, jnp.float32)
    for i in range(CH // 16):
        ones_v[pl.ds(i * 16, 16)] = one16

    @pl.when(s == 0)
    def _():
        def fill(i, carry):
            stage_v[pl.ds(i * 16, 16)] = one16
            return carry
        lax.fori_loop(0, N // 16, fill, 0)
        pltpu.sync_copy(stage_v, acc)

    plsc.subcore_barrier()

    def d_start(b, j):
        off = pl.multiple_of(w * EPT + j * CH, 8)
        pltpu.async_copy(dst_hbm.at[pl.ds(off, CH)], idxv[b], semd[b])

    def d_wait(b):
        pltpu.make_async_copy(dst_hbm.at[pl.ds(0, CH)], idxv[b], semd[b]).wait()

    def s_start(b):
        pltpu.async_copy(ones_v, acc.at[idxv[b]], sems[b], add=True)

    def s_wait(b):
        pltpu.make_async_copy(ones_v, acc.at[idxv[b]], sems[b]).wait()

    def chunk_body(j, b, prefetch_wait):
        jn = jnp.minimum(j + K, NCH - 1)
        bn = (b + K) % NB
        if prefetch_wait:
            s_wait(bn)
        d_start(bn, jn)
        d_wait(b)
        s_start(b)

    d_start(0, 0)
    for j in range(NB):
        chunk_body(j, j % NB, prefetch_wait=(j + K >= NB))

    def outer(i, carry):
        for b in range(NB):
            chunk_body(i * NB + b, b, prefetch_wait=True)
        return carry

    lax.fori_loop(1, (NCH - 2) // NB, outer, 0)
    chunk_body(NCH - 2, (NCH - 2) % NB, prefetch_wait=True)
    chunk_body(NCH - 1, (NCH - 1) % NB, prefetch_wait=True)
    s_wait((NCH - 2) % NB)
    s_wait((NCH - 1) % NB)
    d_wait(NCH % NB)
    plsc.subcore_barrier()

    @pl.when(s == 0)
    def _():
        pltpu.sync_copy(acc, stage_v)
        pltpu.sync_copy(stage_v, out_hbm.at[pl.ds(pl.multiple_of(c * N, 8), N)])


@functools.partial(
    pl.kernel,
    out_type=jax.ShapeDtypeStruct((NC, N_PAD, DIM), jnp.float32),
    mesh=_MESH,
    scratch_types=[
        pltpu.VMEM_SHARED((N_PAD, DIM), jnp.float32),
    ]
    + [pltpu.VMEM((CH, DIM), jnp.float32) for _ in range(NR)]
    + [pltpu.VMEM((CH,), jnp.int32) for _ in range(NI)]
    + [pltpu.VMEM((CH,), jnp.int32) for _ in range(NI)]
    + [pltpu.SemaphoreType.DMA for _ in range(2 * NR + 2 * NI)],
)
def _scatter_kernel(src_hbm, dst_hbm, y_hbm, out_hbm, acc, *ring):
    rows = ring[:NR]
    o = NR
    srcv = ring[o:o + NI]; o += NI
    dstv = ring[o:o + NI]; o += NI
    semg = ring[o:o + NR]; o += NR
    sems = ring[o:o + NR]; o += NR
    semi = ring[o:o + NI]; o += NI
    semd = ring[o:o + NI]
    c = lax.axis_index("c")
    s = lax.axis_index("s")
    w = c * NS + s
    r0 = pl.multiple_of(s * RPT, 8)
    pltpu.sync_copy(y_hbm.at[pl.ds(r0, RPT)], acc.at[pl.ds(r0, RPT)])
    plsc.subcore_barrier()

    def eoff(j):
        return pl.multiple_of(w * EPT + j * CH, 8)

    def i_start(j, m):
        b = m % NI
        pltpu.async_copy(src_hbm.at[pl.ds(eoff(j), CH)], srcv[b], semi[b])

    def i_wait(m):
        b = m % NI
        pltpu.make_async_copy(src_hbm.at[pl.ds(0, CH)], srcv[b], semi[b]).wait()

    def d_start(j, m):
        b = m % NI
        pltpu.async_copy(dst_hbm.at[pl.ds(eoff(j), CH)], dstv[b], semd[b])

    def d_wait(m):
        b = m % NI
        pltpu.make_async_copy(dst_hbm.at[pl.ds(0, CH)], dstv[b], semd[b]).wait()

    def g_start(m):
        pltpu.async_copy(y_hbm.at[srcv[m % NI]], rows[m % NR], semg[m % NR])

    def g_wait(m):
        b = m % NR
        pltpu.make_async_copy(y_hbm.at[srcv[0]], rows[b], semg[b]).wait()

    def s_start(m):
        pltpu.async_copy(rows[m % NR], acc.at[dstv[m % NI]], sems[m % NR], add=True)

    def s_wait(m):
        b = m % NR
        pltpu.make_async_copy(rows[b], acc.at[dstv[0]], sems[b]).wait()

    def chunk_body(j, m, prev3=True, next2=True, next1=True):
        if prev3:
            s_wait(m - 3)
        if next2:
            i_start(j + 2, m + 2)
            d_start(j + 2, m + 2)
        if next1:
            i_wait(m + 1)
            g_start(m + 1)
        g_wait(m)
        d_wait(m)
        s_start(m)

    i_start(0, 0); d_start(0, 0); i_start(1, 1); d_start(1, 1)
    i_wait(0); g_start(0)
    for j in range(3):
        chunk_body(j, j, prev3=False)

    def outer(i, carry):
        j0 = 3 + i * 8
        for t in range(8):
            chunk_body(j0 + t, 3 + t)
        return carry

    lax.fori_loop(0, (NCH - 5) // 8, outer, 0)
    chunk_body(NCH - 2, NCH - 2, next2=False)
    chunk_body(NCH - 1, NCH - 1, next2=False, next1=False)
    for m in (NCH - 3, NCH - 2, NCH - 1):
        s_wait(m)
    plsc.subcore_barrier()
    pltpu.sync_copy(acc.at[pl.ds(r0, RPT)], out_hbm.at[c, pl.ds(r0, RPT)])


def _mlp_prep_body(f_ref, p_ref, w1_ref, b1_ref, w2_ref, b2_ref, deg_ref,
                   w_ref, xn_ref, y_ref):
    h0 = jnp.dot(f_ref[...], w1_ref[...], preferred_element_type=jnp.float32)
    h0 = h0 + b1_ref[...]
    h0 = jnp.where(h0 >= 0, h0, 0.01 * h0)
    temp = jnp.dot(h0, w2_ref[...], preferred_element_type=jnp.float32) + b2_ref[...]
    x = jnp.concatenate([p_ref[...], temp], axis=0)
    n2 = jnp.sum(x * x, axis=1, keepdims=True)
    nrm = jnp.maximum(jnp.sqrt(n2), 1e-12)
    xn = x / nrm
    xn_ref[...] = xn
    y_ref[0:N, :] = (
        jnp.dot(xn, w_ref[...], preferred_element_type=jnp.float32) * _dinv_col(deg_ref)
    )
    y_ref[N:N_PAD, :] = jnp.zeros((N_PAD - N, DIM), jnp.float32)


def _dinv_col(deg_ref):
    dsum = deg_ref[:, 0:1] + deg_ref[:, 1:2] - 1.0
    return lax.rsqrt(dsum)


def _mid_body(s_ref, y_ref, deg_ref, w_ref, b_ref, h_ref, y2_ref):
    dinv = _dinv_col(deg_ref)
    y = y_ref[0:N, :]
    ssum = s_ref[0, 0:N, :] + s_ref[1, 0:N, :] - y
    h = dinv * ssum + b_ref[...]
    h_ref[...] = h
    y2_ref[0:N, :] = (
        jnp.dot(h, w_ref[...], preferred_element_type=jnp.float32) * dinv
    )
    y2_ref[N:N_PAD, :] = jnp.zeros((N_PAD - N, DIM), jnp.float32)


def _fin_body(s_ref, y2_ref, h_ref, xn_ref, deg_ref, b_ref, out_ref):
    dinv = _dinv_col(deg_ref)
    y2 = y2_ref[0:N, :]
    h1 = dinv * (s_ref[0, 0:N, :] + s_ref[1, 0:N, :] - y2) + b_ref[...]
    out_ref[...] = xn_ref[...] + h_ref[...] + h1


def _f32(*shape):
    return jax.ShapeDtypeStruct(shape, jnp.float32)


def kernel(edge_index, features, preference, W_mlp, b_mlp, W_mlp1, b_mlp1, W_conv, b_conv):
    src_f = edge_index[0].astype(jnp.int32)
    dst_f = edge_index[1].astype(jnp.int32)

    deg2 = _deg_kernel(dst_f).reshape(NC, N)
    degT = deg2.T

    xn, y1 = pl.pallas_call(
        _mlp_prep_body, out_shape=(_f32(N, DIM), _f32(N_PAD, DIM))
    )(features, preference, W_mlp.T, b_mlp.reshape(1, -1), W_mlp1.T,
      b_mlp1.reshape(1, -1), degT, W_conv)

    s1 = _scatter_kernel(src_f, dst_f, y1)
    h, y2 = pl.pallas_call(_mid_body, out_shape=(_f32(N, DIM), _f32(N_PAD, DIM)))(
        s1, y1, degT, W_conv, b_conv.reshape(1, -1)
    )

    s2 = _scatter_kernel(src_f, dst_f, y2)
    x_hat = pl.pallas_call(_fin_body, out_shape=_f32(N, DIM))(
        s2, y2, h, xn, degT, b_conv.reshape(1, -1)
    )
    return (x_hat, preference)


# speedup vs baseline: 1.2652x; 1.0566x over previous
import functools

import jax
import jax.numpy as jnp
from jax import lax
from jax.experimental import pallas as pl
from jax.experimental.pallas import tpu as pltpu
from jax.experimental.pallas import tpu_sc as plsc

NUM_USER = 2000
NUM_ITEM = 8000
DIM = 128
N = NUM_USER + NUM_ITEM
E = 320000
NC, NS = 2, 16
NW = NC * NS
EPT = E // NW
CH = 80
NCH = EPT // CH
ND = 6
KD = 3
NR = 4
NI = 8
N_PAD = 10112
RPT = N_PAD // NS

_MESH = plsc.VectorSubcoreMesh(
    core_axis_name="c", subcore_axis_name="s", num_cores=NC, num_subcores=NS
)


@functools.partial(
    pl.kernel,
    out_type=jax.ShapeDtypeStruct((NC * N,), jnp.float32),
    mesh=_MESH,
    scratch_types=[
        pltpu.VMEM((CH,), jnp.float32),
        pltpu.VMEM((N,), jnp.float32),
        pltpu.VMEM_SHARED((N,), jnp.float32),
    ]
    + [pltpu.VMEM((CH,), jnp.int32) for _ in range(ND)]
    + [pltpu.SemaphoreType.DMA for _ in range(2 * ND)],
)
def _deg_kernel(dst_hbm, out_hbm, ones_v, stage_v, acc, *ring):
    idxv = ring[:ND]
    semd = ring[ND:2 * ND]
    sems = ring[2 * ND:]
    c = lax.axis_index("c")
    s = lax.axis_index("s")
    w = c * NS + s
    one16 = jnp.full((16,), 1.0, jnp.float32)
    for i in range(CH // 16):
        ones_v[pl.ds(i * 16, 16)] = one16

    @pl.when(s == 0)
    def _():
        def fill(i, carry):
            stage_v[pl.ds(i * 16, 16)] = one16
            return carry
        lax.fori_loop(0, N // 16, fill, 0)
        pltpu.sync_copy(stage_v, acc)

    plsc.subcore_barrier()

    def d_start(b, j):
        b = b % ND
        off = pl.multiple_of(w * EPT + j * CH, 8)
        pltpu.async_copy(dst_hbm.at[pl.ds(off, CH)], idxv[b], semd[b])

    def d_wait(b):
        pltpu.make_async_copy(dst_hbm.at[pl.ds(0, CH)], idxv[b], semd[b]).wait()

    def s_start(b):
        pltpu.async_copy(ones_v, acc.at[idxv[b]], sems[b], add=True)

    def s_wait(b):
        pltpu.make_async_copy(ones_v, acc.at[idxv[b]], sems[b]).wait()

    def chunk_body(j, m, prev=True):
        jn = jnp.minimum(j + KD, NCH - 1)
        bn = (m + KD) % ND
        if prev:
            s_wait(bn)
        d_start(bn, jn)
        d_wait(m % ND)
        s_start(m % ND)

    for b in range(KD):
        d_start(b, b)
    for j in range(5):
        chunk_body(j, j, prev=(j + KD >= ND))

    def outer(i, carry):
        j0 = 5 + i * 6
        for t in range(6):
            chunk_body(j0 + t, 5 + t)
        return carry

    lax.fori_loop(0, (NCH - 5) // 6, outer, 0)
    for m in (NCH - 3, NCH - 2, NCH - 1):
        s_wait(m % ND)
    for m in (NCH, NCH + 1, NCH + 2):
        d_wait(m % ND)
    plsc.subcore_barrier()

    @pl.when(s == 0)
    def _():
        pltpu.sync_copy(acc, stage_v)
        pltpu.sync_copy(stage_v, out_hbm.at[pl.ds(pl.multiple_of(c * N, 8), N)])


@functools.partial(
    pl.kernel,
    out_type=jax.ShapeDtypeStruct((NC, N_PAD, DIM), jnp.float32),
    mesh=_MESH,
    scratch_types=[
        pltpu.VMEM_SHARED((N_PAD, DIM), jnp.float32),
    ]
    + [pltpu.VMEM((CH, DIM), jnp.float32) for _ in range(NR)]
    + [pltpu.VMEM((CH,), jnp.int32) for _ in range(NI)]
    + [pltpu.VMEM((CH,), jnp.int32) for _ in range(NI)]
    + [pltpu.SemaphoreType.DMA for _ in range(2 * NR + 2 * NI)],
)
def _scatter_kernel(src_hbm, dst_hbm, y_hbm, out_hbm, acc, *ring):
    rows = ring[:NR]
    o = NR
    srcv = ring[o:o + NI]; o += NI
    dstv = ring[o:o + NI]; o += NI
    semg = ring[o:o + NR]; o += NR
    sems = ring[o:o + NR]; o += NR
    semi = ring[o:o + NI]; o += NI
    semd = ring[o:o + NI]
    c = lax.axis_index("c")
    s = lax.axis_index("s")
    w = c * NS + s
    r0 = pl.multiple_of(s * RPT, 8)
    pltpu.sync_copy(y_hbm.at[pl.ds(r0, RPT)], acc.at[pl.ds(r0, RPT)])
    plsc.subcore_barrier()

    def eoff(j):
        return pl.multiple_of(w * EPT + j * CH, 8)

    def i_start(j, m):
        b = m % NI
        pltpu.async_copy(src_hbm.at[pl.ds(eoff(j), CH)], srcv[b], semi[b])

    def i_wait(m):
        b = m % NI
        pltpu.make_async_copy(src_hbm.at[pl.ds(0, CH)], srcv[b], semi[b]).wait()

    def d_start(j, m):
        b = m % NI
        pltpu.async_copy(dst_hbm.at[pl.ds(eoff(j), CH)], dstv[b], semd[b])

    def d_wait(m):
        b = m % NI
        pltpu.make_async_copy(dst_hbm.at[pl.ds(0, CH)], dstv[b], semd[b]).wait()

    def g_start(m):
        pltpu.async_copy(y_hbm.at[srcv[m % NI]], rows[m % NR], semg[m % NR])

    def g_wait(m):
        b = m % NR
        pltpu.make_async_copy(y_hbm.at[srcv[0]], rows[b], semg[b]).wait()

    def s_start(m):
        pltpu.async_copy(rows[m % NR], acc.at[dstv[m % NI]], sems[m % NR], add=True)

    def s_wait(m):
        b = m % NR
        pltpu.make_async_copy(rows[b], acc.at[dstv[0]], sems[b]).wait()

    def chunk_body(j, m, prev3=True, next2=True, next1=True):
        if prev3:
            s_wait(m - 3)
        if next2:
            i_start(j + 2, m + 2)
            d_start(j + 2, m + 2)
        if next1:
            i_wait(m + 1)
            g_start(m + 1)
        g_wait(m)
        d_wait(m)
        s_start(m)

    i_start(0, 0); d_start(0, 0); i_start(1, 1); d_start(1, 1)
    i_wait(0); g_start(0)
    for j in range(3):
        chunk_body(j, j, prev3=False)

    def outer(i, carry):
        j0 = 3 + i * 8
        for t in range(8):
            chunk_body(j0 + t, 3 + t)
        return carry

    lax.fori_loop(0, (NCH - 5) // 8, outer, 0)
    chunk_body(NCH - 2, NCH - 2, next2=False)
    chunk_body(NCH - 1, NCH - 1, next2=False, next1=False)
    for m in (NCH - 3, NCH - 2, NCH - 1):
        s_wait(m)
    plsc.subcore_barrier()
    pltpu.sync_copy(acc.at[pl.ds(r0, RPT)], out_hbm.at[c, pl.ds(r0, RPT)])


def _mlp_body(f_ref, p_ref, w1_ref, b1_ref, w2_ref, b2_ref, out_ref):
    h0 = jnp.dot(f_ref[...], w1_ref[...], preferred_element_type=jnp.float32)
    h0 = h0 + b1_ref[...]
    h0 = jnp.where(h0 >= 0, h0, 0.01 * h0)
    out_ref[0:NUM_USER, :] = p_ref[...]
    out_ref[NUM_USER:N, :] = (
        jnp.dot(h0, w2_ref[...], preferred_element_type=jnp.float32) + b2_ref[...]
    )


def _dinv_col(deg_ref):
    dsum = deg_ref[:, 0:1] + deg_ref[:, 1:2] - 1.0
    return lax.rsqrt(dsum)


def _prep_body(x_ref, deg_ref, w_ref, xn_ref, y_ref):
    x = x_ref[...]
    n2 = jnp.sum(x * x, axis=1, keepdims=True)
    nrm = jnp.maximum(jnp.sqrt(n2), 1e-12)
    xn = x / nrm
    xn_ref[...] = xn
    y_ref[0:N, :] = (
        jnp.dot(xn, w_ref[...], preferred_element_type=jnp.float32) * _dinv_col(deg_ref)
    )
    y_ref[N:N_PAD, :] = jnp.zeros((N_PAD - N, DIM), jnp.float32)


def _mid_body(s_ref, y_ref, deg_ref, w_ref, b_ref, h_ref, y2_ref):
    dinv = _dinv_col(deg_ref)
    y = y_ref[0:N, :]
    ssum = s_ref[0, 0:N, :] + s_ref[1, 0:N, :] - y
    h = dinv * ssum + b_ref[...]
    h_ref[...] = h
    y2_ref[0:N, :] = (
        jnp.dot(h, w_ref[...], preferred_element_type=jnp.float32) * dinv
    )
    y2_ref[N:N_PAD, :] = jnp.zeros((N_PAD - N, DIM), jnp.float32)


def _fin_body(s_ref, y2_ref, h_ref, xn_ref, deg_ref, b_ref, out_ref):
    dinv = _dinv_col(deg_ref)
    y2 = y2_ref[0:N, :]
    h1 = dinv * (s_ref[0, 0:N, :] + s_ref[1, 0:N, :] - y2) + b_ref[...]
    out_ref[...] = xn_ref[...] + h_ref[...] + h1


def _f32(*shape):
    return jax.ShapeDtypeStruct(shape, jnp.float32)


def kernel(edge_index, features, preference, W_mlp, b_mlp, W_mlp1, b_mlp1, W_conv, b_conv):
    src_f = edge_index[0].astype(jnp.int32)
    dst_f = edge_index[1].astype(jnp.int32)

    deg2 = _deg_kernel(dst_f).reshape(NC, N)
    degT = deg2.T

    xcat = pl.pallas_call(_mlp_body, out_shape=_f32(N, DIM))(
        features, preference, W_mlp.T, b_mlp.reshape(1, -1), W_mlp1.T,
        b_mlp1.reshape(1, -1)
    )

    xn, y1 = pl.pallas_call(_prep_body, out_shape=(_f32(N, DIM), _f32(N_PAD, DIM)))(
        xcat, degT, W_conv
    )

    s1 = _scatter_kernel(src_f, dst_f, y1)
    h, y2 = pl.pallas_call(_mid_body, out_shape=(_f32(N, DIM), _f32(N_PAD, DIM)))(
        s1, y1, degT, W_conv, b_conv.reshape(1, -1)
    )

    s2 = _scatter_kernel(src_f, dst_f, y2)
    x_hat = pl.pallas_call(_fin_body, out_shape=_f32(N, DIM))(
        s2, y2, h, xn, degT, b_conv.reshape(1, -1)
    )
    return (x_hat, preference)
